# Initial kernel scaffold; baseline (speedup 1.0000x reference)
import jax
import jax.numpy as jnp
from jax.experimental import pallas as pl


def kernel(x, edge_index, W0, b0, W1, b1, W2, b2, W3, b3, W4, b4, W5, b5, Wl, bl):
    raise NotImplementedError("write your pallas kernel here")


# speedup vs baseline: 29.0118x; 29.0118x over previous
import functools

import jax
import jax.numpy as jnp
from jax import lax
from jax.experimental import pallas as pl
from jax.experimental.pallas import tpu as pltpu
from jax.experimental.pallas import tpu_sc as plsc

N_NODES = 100000
N_EDGES = 3200000
IN_FEAT = 128
N_UNITS = 16
N_LAYERS = 6
OUT_FEAT = 64

CHUNK = 128
K = 8
NSUP = 98
CPT = K * NSUP
NW = 32
TOT_CHUNKS = CPT * NW
PAD_E = TOT_CHUNKS * CHUNK

RB = 1568
R = 100352
RPT = R // 16


def _sc_agg_body(hw, src2, dst2, out, agg_sp, src_v, dst_v, msgs_v, zero_v,
                 sem_g):
    c = lax.axis_index("c")
    s = lax.axis_index("s")
    wid = c * 16 + s

    def zrow(r, carry):
        zero_v[r, :] = jnp.zeros((16,), jnp.float32)
        return carry

    lax.fori_loop(0, 128, zrow, 0)

    def zblk(j, carry):
        pltpu.sync_copy(zero_v, agg_sp.at[pl.ds(s * RPT + j * 128, 128)])
        return carry

    lax.fori_loop(0, RPT // 128, zblk, 0)
    plsc.subcore_barrier()

    base = wid * CPT

    def group(i, carry):
        cb = base + i * K
        pltpu.sync_copy(src2.at[pl.ds(cb, K)], src_v)
        pltpu.sync_copy(dst2.at[pl.ds(cb, K)], dst_v)
        gathers = [
            pltpu.async_copy(hw.at[src_v.at[j]], msgs_v.at[j], sem_g)
            for j in range(K)
        ]
        for g in gathers:
            g.wait()
        for j in range(K):
            pltpu.sync_copy(msgs_v.at[j], agg_sp.at[dst_v.at[j]], add=True)
        return carry

    lax.fori_loop(0, NSUP, group, 0)
    plsc.subcore_barrier()

    pltpu.sync_copy(agg_sp.at[pl.ds(s * RPT, RPT)],
                    out.at[pl.ds(c * R + s * RPT, RPT)])


_sc_agg = functools.partial(
    pl.kernel,
    out_type=jax.ShapeDtypeStruct((2 * R, N_UNITS), jnp.float32),
    mesh=plsc.VectorSubcoreMesh(core_axis_name="c", subcore_axis_name="s"),
    scratch_types=[
        pltpu.VMEM_SHARED((R, N_UNITS), jnp.float32),
        pltpu.VMEM((K, CHUNK), jnp.int32),
        pltpu.VMEM((K, CHUNK), jnp.int32),
        pltpu.VMEM((K, CHUNK, N_UNITS), jnp.float32),
        pltpu.VMEM((128, N_UNITS), jnp.float32),
        pltpu.SemaphoreType.DMA,
    ],
    compiler_params=pltpu.CompilerParams(use_tc_tiling_on_sc=False),
)(_sc_agg_body)


def _proj_body(x_ref, w_ref, b_ref, hw_ref):
    hw_ref[...] = (
        jnp.dot(x_ref[...], w_ref[...], preferred_element_type=jnp.float32)
        + b_ref[...]
    )


def _layer_body(p0_ref, p1_ref, w_ref, b_ref, h_ref, hw_ref):
    h = jnp.maximum(p0_ref[...] + p1_ref[...], 0.0)
    h_ref[...] = h
    hw_ref[...] = (
        jnp.dot(h, w_ref[...], preferred_element_type=jnp.float32) + b_ref[...]
    )


def _last_layer_body(p0_ref, p1_ref, h_ref):
    h_ref[...] = jnp.maximum(p0_ref[...] + p1_ref[...], 0.0)


def _final_body(h0, h1, h2, h3, h4, h5, wl_ref, bl_ref, out_ref):
    hcat = jnp.concatenate(
        [h0[...], h1[...], h2[...], h3[...], h4[...], h5[...]], axis=1)
    out_ref[...] = (
        jnp.dot(hcat, wl_ref[...], preferred_element_type=jnp.float32)
        + bl_ref[...]
    )


def _row_spec(cols):
    return pl.BlockSpec((RB, cols), lambda i: (i, 0))


def _full_spec(shape):
    return pl.BlockSpec(shape, lambda i: tuple(0 for _ in shape))


_GRID = R // RB

_proj = pl.pallas_call(
    _proj_body,
    grid=(_GRID,),
    in_specs=[
        _row_spec(IN_FEAT),
        _full_spec((IN_FEAT, N_UNITS)),
        _full_spec((1, N_UNITS)),
    ],
    out_specs=_row_spec(N_UNITS),
    out_shape=jax.ShapeDtypeStruct((R, N_UNITS), jnp.float32),
)


def _part_specs():
    return [
        pl.BlockSpec((RB, N_UNITS), lambda i: (i, 0)),
        pl.BlockSpec((RB, N_UNITS), lambda i: (i + _GRID, 0)),
    ]


_layer = pl.pallas_call(
    _layer_body,
    grid=(_GRID,),
    in_specs=_part_specs() + [
        _full_spec((N_UNITS, N_UNITS)),
        _full_spec((1, N_UNITS)),
    ],
    out_specs=[_row_spec(N_UNITS), _row_spec(N_UNITS)],
    out_shape=[
        jax.ShapeDtypeStruct((R, N_UNITS), jnp.float32),
        jax.ShapeDtypeStruct((R, N_UNITS), jnp.float32),
    ],
)

_last_layer = pl.pallas_call(
    _last_layer_body,
    grid=(_GRID,),
    in_specs=_part_specs(),
    out_specs=_row_spec(N_UNITS),
    out_shape=jax.ShapeDtypeStruct((R, N_UNITS), jnp.float32),
)

_final = pl.pallas_call(
    _final_body,
    grid=(_GRID,),
    in_specs=[_row_spec(N_UNITS)] * N_LAYERS + [
        _full_spec((N_LAYERS * N_UNITS, OUT_FEAT)),
        _full_spec((1, OUT_FEAT)),
    ],
    out_specs=_row_spec(OUT_FEAT),
    out_shape=jax.ShapeDtypeStruct((N_NODES, OUT_FEAT), jnp.float32),
)


def kernel(x, edge_index, W0, b0, W1, b1, W2, b2, W3, b3, W4, b4, W5, b5,
           Wl, bl):
    src = edge_index[0]
    dst = edge_index[1]
    npad = PAD_E - N_EDGES
    src2 = jnp.concatenate(
        [src, jnp.zeros((npad,), jnp.int32)]).reshape(TOT_CHUNKS, CHUNK)
    dst2 = jnp.concatenate(
        [dst, jnp.full((npad,), N_NODES, jnp.int32)]).reshape(
            TOT_CHUNKS, CHUNK)

    Ws = [W1, W2, W3, W4, W5]
    bs = [b1.reshape(1, N_UNITS), b2.reshape(1, N_UNITS),
          b3.reshape(1, N_UNITS), b4.reshape(1, N_UNITS),
          b5.reshape(1, N_UNITS)]

    hw = _proj(x, W0, b0.reshape(1, N_UNITS))
    hs = []
    for i in range(N_LAYERS):
        parts = _sc_agg(hw, src2, dst2)
        if i < N_LAYERS - 1:
            h, hw = _layer(parts, parts, Ws[i], bs[i])
        else:
            h = _last_layer(parts, parts)
        hs.append(h)

    return _final(hs[0], hs[1], hs[2], hs[3], hs[4], hs[5],
                  Wl, bl.reshape(1, OUT_FEAT))


# speedup vs baseline: 41.5335x; 1.4316x over previous
import functools

import jax
import jax.numpy as jnp
from jax import lax
from jax.experimental import pallas as pl
from jax.experimental.pallas import tpu as pltpu
from jax.experimental.pallas import tpu_sc as plsc

N_NODES = 100000
N_EDGES = 3200000
IN_FEAT = 128
N_UNITS = 16
N_LAYERS = 6
OUT_FEAT = 64

CHUNK = 128
K = 8
NSUP = 98
CPT = K * NSUP
NW = 32
TOT_CHUNKS = CPT * NW
PAD_E = TOT_CHUNKS * CHUNK

RB = 1568
R = 100352
RPT = R // 16


def _sc_agg_body(hw, src2, dst2, out, agg_sp, src_a, dst_a, src_b, dst_b,
                 msgs_v, zero_v, sem_i, sem_g, sem_s):
    c = lax.axis_index("c")
    s = lax.axis_index("s")
    wid = c * 16 + s

    def zrow(r, carry):
        zero_v[r, :] = jnp.zeros((16,), jnp.float32)
        return carry

    lax.fori_loop(0, 128, zrow, 0)

    def zblk(j, carry):
        pltpu.sync_copy(zero_v, agg_sp.at[pl.ds(s * RPT + j * 128, 128)])
        return carry

    lax.fori_loop(0, RPT // 128, zblk, 0)
    plsc.subcore_barrier()

    base = wid * CPT

    pltpu.sync_copy(src2.at[pl.ds(base, K)], src_a)
    pltpu.sync_copy(dst2.at[pl.ds(base, K)], dst_a)
    pltpu.async_copy(src2.at[pl.ds(base + K, K)], src_b, sem_i)
    pltpu.async_copy(dst2.at[pl.ds(base + K, K)], dst_b, sem_i)

    def half(sv, dv, nsv, ndv, cb_next, do_wait, cb_pf, do_pf):
        gathers = [
            pltpu.async_copy(hw.at[sv.at[j]], msgs_v.at[j], sem_g)
            for j in range(K)
        ]
        for g in gathers:
            g.wait()
        scatters = [
            pltpu.async_copy(msgs_v.at[j], agg_sp.at[dv.at[j]], sem_s,
                             add=True)
            for j in range(K)
        ]

        def wait_next():
            pltpu.make_async_copy(
                src2.at[pl.ds(cb_next, K)], nsv, sem_i).wait()
            pltpu.make_async_copy(
                dst2.at[pl.ds(cb_next, K)], ndv, sem_i).wait()

        if do_wait is None:
            wait_next()
        else:
            pl.when(do_wait)(wait_next)
        for sc_ in scatters:
            sc_.wait()

        @pl.when(do_pf)
        def _():
            pltpu.async_copy(src2.at[pl.ds(cb_pf, K)], sv, sem_i)
            pltpu.async_copy(dst2.at[pl.ds(cb_pf, K)], dv, sem_i)

    NP = NSUP // 2
    _MAXCB = TOT_CHUNKS - K

    def pair(i, carry):
        cb_a = base + 2 * i * K
        not_last = i < NP - 1
        half(src_a, dst_a, src_b, dst_b, cb_a + K, None,
             jnp.minimum(cb_a + 2 * K, _MAXCB), not_last)
        half(src_b, dst_b, src_a, dst_a,
             jnp.minimum(cb_a + 2 * K, _MAXCB), not_last,
             jnp.minimum(cb_a + 3 * K, _MAXCB), not_last)
        return carry

    lax.fori_loop(0, NP, pair, 0)
    plsc.subcore_barrier()

    pltpu.sync_copy(agg_sp.at[pl.ds(s * RPT, RPT)],
                    out.at[pl.ds(c * R + s * RPT, RPT)])


_sc_agg = functools.partial(
    pl.kernel,
    out_type=jax.ShapeDtypeStruct((2 * R, N_UNITS), jnp.float32),
    mesh=plsc.VectorSubcoreMesh(core_axis_name="c", subcore_axis_name="s"),
    scratch_types=[
        pltpu.VMEM_SHARED((R, N_UNITS), jnp.float32),
        pltpu.VMEM((K, CHUNK), jnp.int32),
        pltpu.VMEM((K, CHUNK), jnp.int32),
        pltpu.VMEM((K, CHUNK), jnp.int32),
        pltpu.VMEM((K, CHUNK), jnp.int32),
        pltpu.VMEM((K, CHUNK, N_UNITS), jnp.float32),
        pltpu.VMEM((128, N_UNITS), jnp.float32),
        pltpu.SemaphoreType.DMA,
        pltpu.SemaphoreType.DMA,
        pltpu.SemaphoreType.DMA,
    ],
    compiler_params=pltpu.CompilerParams(use_tc_tiling_on_sc=False),
)(_sc_agg_body)


def _proj_body(x_ref, w_ref, b_ref, hw_ref):
    hw_ref[...] = (
        jnp.dot(x_ref[...], w_ref[...], preferred_element_type=jnp.float32)
        + b_ref[...]
    )


def _layer_body(p0_ref, p1_ref, w_ref, b_ref, h_ref, hw_ref):
    h = jnp.maximum(p0_ref[...] + p1_ref[...], 0.0)
    h_ref[...] = h
    hw_ref[...] = (
        jnp.dot(h, w_ref[...], preferred_element_type=jnp.float32) + b_ref[...]
    )


def _last_layer_body(p0_ref, p1_ref, h_ref):
    h_ref[...] = jnp.maximum(p0_ref[...] + p1_ref[...], 0.0)


def _final_body(h0, h1, h2, h3, h4, h5, wl_ref, bl_ref, out_ref):
    hcat = jnp.concatenate(
        [h0[...], h1[...], h2[...], h3[...], h4[...], h5[...]], axis=1)
    out_ref[...] = (
        jnp.dot(hcat, wl_ref[...], preferred_element_type=jnp.float32)
        + bl_ref[...]
    )


def _row_spec(cols):
    return pl.BlockSpec((RB, cols), lambda i: (i, 0))


def _full_spec(shape):
    return pl.BlockSpec(shape, lambda i: tuple(0 for _ in shape))


_GRID = R // RB

_proj = pl.pallas_call(
    _proj_body,
    grid=(_GRID,),
    in_specs=[
        _row_spec(IN_FEAT),
        _full_spec((IN_FEAT, N_UNITS)),
        _full_spec((1, N_UNITS)),
    ],
    out_specs=_row_spec(N_UNITS),
    out_shape=jax.ShapeDtypeStruct((R, N_UNITS), jnp.float32),
)


def _part_specs():
    return [
        pl.BlockSpec((RB, N_UNITS), lambda i: (i, 0)),
        pl.BlockSpec((RB, N_UNITS), lambda i: (i + _GRID, 0)),
    ]


_layer = pl.pallas_call(
    _layer_body,
    grid=(_GRID,),
    in_specs=_part_specs() + [
        _full_spec((N_UNITS, N_UNITS)),
        _full_spec((1, N_UNITS)),
    ],
    out_specs=[_row_spec(N_UNITS), _row_spec(N_UNITS)],
    out_shape=[
        jax.ShapeDtypeStruct((R, N_UNITS), jnp.float32),
        jax.ShapeDtypeStruct((R, N_UNITS), jnp.float32),
    ],
)

_last_layer = pl.pallas_call(
    _last_layer_body,
    grid=(_GRID,),
    in_specs=_part_specs(),
    out_specs=_row_spec(N_UNITS),
    out_shape=jax.ShapeDtypeStruct((R, N_UNITS), jnp.float32),
)

_final = pl.pallas_call(
    _final_body,
    grid=(_GRID,),
    in_specs=[_row_spec(N_UNITS)] * N_LAYERS + [
        _full_spec((N_LAYERS * N_UNITS, OUT_FEAT)),
        _full_spec((1, OUT_FEAT)),
    ],
    out_specs=_row_spec(OUT_FEAT),
    out_shape=jax.ShapeDtypeStruct((N_NODES, OUT_FEAT), jnp.float32),
)


def kernel(x, edge_index, W0, b0, W1, b1, W2, b2, W3, b3, W4, b4, W5, b5,
           Wl, bl):
    src = edge_index[0]
    dst = edge_index[1]
    npad = PAD_E - N_EDGES
    it = jnp.arange(npad, dtype=jnp.int32)
    src2 = jnp.concatenate(
        [src, (it * 61) % N_NODES]).reshape(TOT_CHUNKS, CHUNK)
    dst2 = jnp.concatenate(
        [dst, N_NODES + it % (R - N_NODES)]).reshape(TOT_CHUNKS, CHUNK)

    Ws = [W1, W2, W3, W4, W5]
    bs = [b1.reshape(1, N_UNITS), b2.reshape(1, N_UNITS),
          b3.reshape(1, N_UNITS), b4.reshape(1, N_UNITS),
          b5.reshape(1, N_UNITS)]

    hw = _proj(x, W0, b0.reshape(1, N_UNITS))
    hs = []
    for i in range(N_LAYERS):
        parts = _sc_agg(hw, src2, dst2)
        if i < N_LAYERS - 1:
            h, hw = _layer(parts, parts, Ws[i], bs[i])
        else:
            h = _last_layer(parts, parts)
        hs.append(h)

    return _final(hs[0], hs[1], hs[2], hs[3], hs[4], hs[5],
                  Wl, bl.reshape(1, OUT_FEAT))
